# Initial kernel scaffold; baseline (speedup 1.0000x reference)
import jax
import jax.numpy as jnp
from jax.experimental import pallas as pl


def kernel(x, edge_index, edge_weight, batch, W_rel0, b_rel0, W_root0, W_rel1, b_rel1, W_root1, W_rel2, b_rel2, W_root2, lin1_W, lin1_b, lin2_W, lin2_b):
    raise NotImplementedError("write your pallas kernel here")


# speedup vs baseline: 3.3465x; 3.3465x over previous
import functools

import jax
import jax.numpy as jnp
from jax import lax
from jax.experimental import pallas as pl
from jax.experimental.pallas import tpu as pltpu
from jax.experimental.pallas import tpu_sc as plsc

_N = 10000
_D = 128
_E = 320000
_G = 64
_OUT = 64
_CH = 64
_NCH = 160
_EP = 32 * _NCH * _CH
_RPT = 624
_BM = 2000


def _sc_agg_body(h_hbm, e_hbm, out_hbm,
                 ering, r0, r1, r2, r3, r4, acc,
                 g0, g1, g2, g3, g4, s0, s1, s2, s3, s4,
                 i0, i1, i2, i3, i4):
    c = lax.axis_index("c")
    s = lax.axis_index("s")
    bufs = (r0, r1, r2, r3, r4)
    gsems = (g0, g1, g2, g3, g4)
    ssems = (s0, s1, s2, s3, s4)
    isems = (i0, i1, i2, i3, i4)

    zero16 = jnp.zeros((16,), jnp.float32)

    @plsc.parallel_loop(0, _CH)
    def _zrow(r):
        for cc in range(8):
            r0[r, pl.ds(cc * 16, 16)] = zero16

    base = s * _RPT
    for k in range(9):
        pltpu.sync_copy(r0, acc.at[pl.ds(base + k * _CH, _CH)])
    pltpu.sync_copy(r0.at[pl.ds(0, 48)], acc.at[pl.ds(base + 576, 48)])

    @pl.when(s == 15)
    def _ztail():
        pltpu.sync_copy(r0.at[pl.ds(0, 16)], acc.at[pl.ds(9984, 16)])

    plsc.subcore_barrier()

    def idx_load(j, b):
        pltpu.async_copy(e_hbm.at[c, s, j], ering.at[pl.ds(3 * b, 3)],
                         isems[b])

    def wait_idx(b):
        pltpu.make_async_copy(e_hbm.at[c, s, 0], ering.at[pl.ds(3 * b, 3)],
                              isems[b]).wait()

    def gather(j, b):
        pltpu.async_copy(h_hbm.at[ering.at[3 * b]], bufs[b], gsems[b])

    def wait_gather(b):
        pltpu.make_async_copy(h_hbm.at[ering.at[3 * b]], bufs[b],
                              gsems[b]).wait()

    def scat(b):
        pltpu.async_copy(bufs[b], acc.at[ering.at[3 * b + 1]], ssems[b],
                         add=True)

    def wait_scat(b):
        pltpu.make_async_copy(bufs[b], acc.at[ering.at[3 * b + 1]],
                              ssems[b]).wait()

    def step(jj, pos, wait_prev):
        b = pos
        bf = (pos + 3) % 5
        bg = (pos + 2) % 5
        rv = bufs[b]
        wref = ering.at[3 * b + 2]
        wait_gather(b)

        @plsc.parallel_loop(0, _CH, unroll=2)
        def _edge(e):
            wv = plsc.bitcast(
                plsc.load_gather(wref, [jnp.full((16,), e, jnp.int32)]),
                jnp.float32)
            for cc in range(8):
                sl = pl.ds(cc * 16, 16)
                rv[e, sl] = rv[e, sl] * wv

        scat(b)
        if wait_prev:
            wait_scat(bf)
        idx_load(jnp.minimum(jj + 3, _NCH - 1), bf)
        wait_idx(bg)
        gather(jnp.minimum(jj + 2, _NCH - 1), bg)

    idx_load(0, 0)
    idx_load(1, 1)
    wait_idx(0)
    wait_idx(1)
    gather(0, 0)
    gather(1, 1)
    idx_load(2, 2)
    for pos in range(5):
        step(pos, pos, pos >= 2)

    def rnd(r, carry):
        for pos in range(5):
            step(5 * r + pos, pos, True)
        return carry

    lax.fori_loop(1, _NCH // 5, rnd, 0)
    wait_scat(3)
    wait_scat(4)
    wait_gather(0)
    wait_gather(1)
    wait_idx(2)
    plsc.subcore_barrier()

    pltpu.sync_copy(acc.at[pl.ds(base, _RPT)],
                    out_hbm.at[c, pl.ds(base, _RPT)])

    @pl.when(s == 15)
    def _wtail():
        pltpu.sync_copy(acc.at[pl.ds(9984, 16)],
                        out_hbm.at[c, pl.ds(9984, 16)])


def _sc_agg(h, edata):
    return pl.kernel(
        _sc_agg_body,
        out_type=jax.ShapeDtypeStruct((2, _N, _D), jnp.float32),
        mesh=plsc.VectorSubcoreMesh(core_axis_name="c", subcore_axis_name="s"),
        scratch_types=(
            [pltpu.VMEM((15, _CH), jnp.int32)]
            + [pltpu.VMEM((_CH, _D), jnp.float32) for _ in range(5)]
            + [pltpu.VMEM_SHARED((_N, _D), jnp.float32)]
            + [pltpu.SemaphoreType.DMA for _ in range(15)]
        ),
        compiler_params=pltpu.CompilerParams(needs_layout_passes=False),
    )(h, edata)


def _tc_layer_body(relu, p_ref, h_ref, wr_ref, wt_ref, b_ref, o_ref):
    agg = p_ref[0] + p_ref[1]
    out = jnp.dot(agg, wr_ref[...], preferred_element_type=jnp.float32)
    out = out + jnp.dot(h_ref[...], wt_ref[...],
                        preferred_element_type=jnp.float32)
    out = out + b_ref[...]
    if relu:
        out = jnp.maximum(out, 0.0)
    o_ref[...] = out


def _tc_layer(p, h, Wr, Wt, b, relu):
    return pl.pallas_call(
        functools.partial(_tc_layer_body, relu),
        grid=(_N // _BM,),
        in_specs=[
            pl.BlockSpec((2, _BM, _D), lambda i: (0, i, 0)),
            pl.BlockSpec((_BM, _D), lambda i: (i, 0)),
            pl.BlockSpec((_D, _D), lambda i: (0, 0)),
            pl.BlockSpec((_D, _D), lambda i: (0, 0)),
            pl.BlockSpec((1, _D), lambda i: (0, 0)),
        ],
        out_specs=pl.BlockSpec((_BM, _D), lambda i: (i, 0)),
        out_shape=jax.ShapeDtypeStruct((_N, _D), jnp.float32),
    )(p, h, Wr, Wt, b)


def _pool_head_body(h_ref, b3_ref, w1_ref, b1_ref, w2_ref, b2_ref, o_ref,
                    sums, cnts):
    i = pl.program_id(0)

    @pl.when(i == 0)
    def _init():
        sums[...] = jnp.zeros_like(sums)
        cnts[...] = jnp.zeros_like(cnts)

    bvec = b3_ref[0]
    oh = (lax.broadcasted_iota(jnp.int32, (_G, _BM), 0) == bvec
          ).astype(jnp.float32)
    sums[...] += jnp.dot(oh, h_ref[...], preferred_element_type=jnp.float32)
    cnts[...] += jnp.broadcast_to(jnp.sum(oh, axis=1, keepdims=True),
                                  (_G, _D))

    @pl.when(i == pl.num_programs(0) - 1)
    def _final():
        pooled = sums[...] / jnp.maximum(cnts[...], 1.0)
        r = jnp.dot(pooled, w1_ref[...], preferred_element_type=jnp.float32)
        r = jnp.maximum(r + b1_ref[...], 0.0)
        o_ref[...] = (jnp.dot(r, w2_ref[...],
                              preferred_element_type=jnp.float32)
                      + b2_ref[...])


def _pool_head(h, batch3, w1, b1, w2, b2):
    return pl.pallas_call(
        _pool_head_body,
        grid=(_N // _BM,),
        in_specs=[
            pl.BlockSpec((_BM, _D), lambda i: (i, 0)),
            pl.BlockSpec((1, 1, _BM), lambda i: (i, 0, 0)),
            pl.BlockSpec((_D, _D), lambda i: (0, 0)),
            pl.BlockSpec((1, _D), lambda i: (0, 0)),
            pl.BlockSpec((_D, _OUT), lambda i: (0, 0)),
            pl.BlockSpec((1, _OUT), lambda i: (0, 0)),
        ],
        out_specs=pl.BlockSpec((_G, _OUT), lambda i: (0, 0)),
        out_shape=jax.ShapeDtypeStruct((_G, _OUT), jnp.float32),
        scratch_shapes=[
            pltpu.VMEM((_G, _D), jnp.float32),
            pltpu.VMEM((_G, _D), jnp.float32),
        ],
    )(h, batch3, w1, b1, w2, b2)


def kernel(x, edge_index, edge_weight, batch, W_rel0, b_rel0, W_root0,
           W_rel1, b_rel1, W_root1, W_rel2, b_rel2, W_root2,
           lin1_W, lin1_b, lin2_W, lin2_b):
    srcs = jnp.pad(edge_index[0], (0, _EP - _E)).reshape(2, 16, _NCH, _CH)
    dsts = jnp.pad(edge_index[1], (0, _EP - _E)).reshape(2, 16, _NCH, _CH)
    wsi = lax.bitcast_convert_type(
        jnp.pad(edge_weight, (0, _EP - _E)), jnp.int32
    ).reshape(2, 16, _NCH, _CH)
    edata = jnp.stack([srcs, dsts, wsi], axis=3)
    batch3 = batch.reshape(_N // _BM, 1, _BM)

    h = x
    layers = [(W_rel0, b_rel0, W_root0, True),
              (W_rel1, b_rel1, W_root1, True),
              (W_rel2, b_rel2, W_root2, False)]
    for Wr, br, Wt, relu in layers:
        p = _sc_agg(h, edata)
        h = _tc_layer(p, h, Wr, Wt, br.reshape(1, _D), relu)

    return _pool_head(h, batch3, lin1_W, lin1_b.reshape(1, _D),
                      lin2_W, lin2_b.reshape(1, _OUT))
